# Initial kernel scaffold; baseline (speedup 1.0000x reference)
import jax
import jax.numpy as jnp
from jax.experimental import pallas as pl


def kernel(tok_ids, emb_table):
    raise NotImplementedError("write your pallas kernel here")


# speedup vs baseline: 4.7480x; 4.7480x over previous
import functools
import math

import jax
import jax.numpy as jnp
from jax import lax
from jax.experimental import pallas as pl
from jax.experimental.pallas import tpu as pltpu
from jax.experimental.pallas import tpu_sc as plsc


def _scale_body(scale, t_ref, o_ref):
    o_ref[...] = t_ref[...] * scale


def _scale_table(table, scale):
    v, d = table.shape
    block = 2000
    assert v % block == 0
    return pl.pallas_call(
        functools.partial(_scale_body, scale),
        grid=(v // block,),
        in_specs=[pl.BlockSpec((block, d), lambda i: (i, 0))],
        out_specs=pl.BlockSpec((block, d), lambda i: (i, 0)),
        out_shape=jax.ShapeDtypeStruct((v, d), table.dtype),
    )(table)


@functools.lru_cache(maxsize=None)
def _make_gather(total, d):
    info = plsc.get_sparse_core_info()
    nc, ns = info.num_cores, info.num_subcores
    nw = nc * ns
    assert total % nw == 0
    per_w = total // nw
    chunk = 128
    assert per_w % chunk == 0
    n_chunks = per_w // chunk
    mesh = plsc.VectorSubcoreMesh(core_axis_name="c", subcore_axis_name="s")

    @functools.partial(
        pl.kernel,
        mesh=mesh,
        out_type=jax.ShapeDtypeStruct((total, d), jnp.float32),
        scratch_types=[
            pltpu.VMEM((chunk,), jnp.int32),
            pltpu.VMEM((chunk, d), jnp.float32),
            pltpu.SemaphoreType.DMA,
        ],
    )
    def gather(idx_hbm, table_hbm, out_hbm, idx_v, rows_v, sem):
        wid = lax.axis_index("s") * nc + lax.axis_index("c")
        base = wid * per_w

        def body(i, carry):
            off = base + i * chunk
            pltpu.sync_copy(idx_hbm.at[pl.ds(off, chunk)], idx_v)
            pltpu.async_copy(table_hbm.at[idx_v], rows_v, sem).wait()
            pltpu.sync_copy(rows_v, out_hbm.at[pl.ds(off, chunk)])
            return carry

        lax.fori_loop(0, n_chunks, body, 0)

    return gather


def kernel(tok_ids, emb_table):
    b, l = tok_ids.shape
    v, d = emb_table.shape
    scaled = _scale_table(emb_table, math.sqrt(float(d)))
    flat = tok_ids.reshape(-1).astype(jnp.int32)
    out = _make_gather(b * l, d)(flat, scaled)
    return out.reshape(b, l, d)


# speedup vs baseline: 7.9195x; 1.6680x over previous
import functools
import math

import jax
import jax.numpy as jnp
from jax import lax
from jax.experimental import pallas as pl
from jax.experimental.pallas import tpu as pltpu
from jax.experimental.pallas import tpu_sc as plsc


def _scale_body(scale, t_ref, o_ref):
    o_ref[...] = t_ref[...] * scale


def _scale_table(table, scale):
    v, d = table.shape
    block = 2000
    assert v % block == 0
    return pl.pallas_call(
        functools.partial(_scale_body, scale),
        grid=(v // block,),
        in_specs=[pl.BlockSpec((block, d), lambda i: (i, 0))],
        out_specs=pl.BlockSpec((block, d), lambda i: (i, 0)),
        out_shape=jax.ShapeDtypeStruct((v, d), table.dtype),
    )(table)


@functools.lru_cache(maxsize=None)
def _make_gather(total, d):
    info = plsc.get_sparse_core_info()
    nc, ns = info.num_cores, info.num_subcores
    nw = nc * ns
    chunk = 128
    nbuf = 4
    assert total % (nw * chunk * nbuf) == 0
    per_w = total // nw
    n_chunks = per_w // chunk
    n_groups = n_chunks // nbuf
    mesh = plsc.VectorSubcoreMesh(core_axis_name="c", subcore_axis_name="s")

    @functools.partial(
        pl.kernel,
        mesh=mesh,
        out_type=jax.ShapeDtypeStruct((total, d), jnp.float32),
        scratch_types=[
            pltpu.VMEM((n_chunks, chunk), jnp.int32),
            pltpu.VMEM((nbuf, chunk, d), jnp.float32),
        ]
        + [pltpu.SemaphoreType.DMA] * (2 * nbuf),
    )
    def gather(idx_hbm, table_hbm, out_hbm, idx_v, rows_v, *sems):
        sg, so = sems[:nbuf], sems[nbuf:]
        wid = lax.axis_index("s") * nc + lax.axis_index("c")
        base = wid * per_w

        pltpu.sync_copy(idx_hbm.at[pl.ds(wid * n_chunks, n_chunks)], idx_v)

        def fire_gather(i, b):
            pltpu.async_copy(table_hbm.at[idx_v.at[i]], rows_v.at[b], sg[b])

        def wait_gather(b):
            pltpu.make_async_copy(
                out_hbm.at[pl.ds(0, chunk)], rows_v.at[b], sg[b]
            ).wait()

        def fire_out(i, b):
            pltpu.async_copy(
                rows_v.at[b], out_hbm.at[pl.ds(base + i * chunk, chunk)], so[b]
            )

        def wait_out(b):
            pltpu.make_async_copy(
                rows_v.at[b], out_hbm.at[pl.ds(0, chunk)], so[b]
            ).wait()

        for b in range(nbuf):
            fire_gather(b, b)

        def body(g, carry):
            for b in range(nbuf):
                wait_gather(b)
                fire_out(g * nbuf + b, b)
            for b in range(nbuf):
                wait_out(b)
                fire_gather((g + 1) * nbuf + b, b)
            return carry

        lax.fori_loop(0, n_groups - 1, body, 0)

        last = (n_groups - 1) * nbuf
        for b in range(nbuf):
            wait_gather(b)
            fire_out(last + b, b)
        for b in range(nbuf):
            wait_out(b)

    return gather


def kernel(tok_ids, emb_table):
    b, l = tok_ids.shape
    v, d = emb_table.shape
    scaled = _scale_table(emb_table, math.sqrt(float(d)))
    flat = tok_ids.reshape(-1, 128).astype(jnp.int32)
    out = _make_gather(b * l, d)(flat, scaled)
    return out.reshape(b, l, d)
